# baseline (speedup 1.0000x reference)
import functools

import jax
import jax.numpy as jnp
from jax import lax
from jax.experimental import pallas as pl
from jax.experimental.pallas import tpu as pltpu
from jax.experimental.pallas import tpu_sc as plsc

_D = 64
_NC = 2
_NS = 16
_NW = _NC * _NS
_BPW = 1024
_CHUNK = 128
_NCHUNK = _BPW // _CHUNK


@functools.lru_cache(maxsize=None)
def _make_gather(b0: int, b1: int):
    wpb = _NW // b0
    bpb = b1 // wpb
    mesh = plsc.VectorSubcoreMesh(core_axis_name="c", subcore_axis_name="s")

    @functools.partial(
        pl.kernel,
        mesh=mesh,
        out_type=jax.ShapeDtypeStruct((b0, b1, _D), jnp.float32),
        scratch_types=[
            pltpu.VMEM((_BPW,), jnp.int32),
            pltpu.VMEM((_BPW, _D), jnp.float32),
            pltpu.SemaphoreType.DMA,
        ],
        compiler_params=pltpu.CompilerParams(use_tc_tiling_on_sc=False),
    )
    def gather(x_hbm, w_hbm, out_hbm, idx_v, rows_v, sem):
        wid = lax.axis_index("s") * _NC + lax.axis_index("c")
        b = wid // wpb
        off = (wid % wpb) * bpb
        pltpu.sync_copy(x_hbm.at[b, pl.ds(off, _BPW)], idx_v)
        copies = [
            pltpu.async_copy(
                w_hbm.at[idx_v.at[pl.ds(c * _CHUNK, _CHUNK)]],
                rows_v.at[pl.ds(c * _CHUNK, _CHUNK)],
                sem,
            )
            for c in range(_NCHUNK)
        ]
        for h in copies:
            h.wait()
        pltpu.sync_copy(rows_v, out_hbm.at[b, pl.ds(off, _BPW)])

    return gather


def kernel(x, weight):
    b0, b1 = x.shape
    return _make_gather(b0, b1)(x.astype(jnp.int32), weight)


# speedup vs baseline: 1.0019x; 1.0019x over previous
import functools

import jax
import jax.numpy as jnp
from jax import lax
from jax.experimental import pallas as pl
from jax.experimental.pallas import tpu as pltpu
from jax.experimental.pallas import tpu_sc as plsc

_D = 64
_NC = 2
_NS = 16
_NW = _NC * _NS
_BPW = 1024
_CHUNK = 128
_NCHUNK = _BPW // _CHUNK
_XT = 128

_MESH = plsc.VectorSubcoreMesh(core_axis_name="c", subcore_axis_name="s")


@functools.lru_cache(maxsize=None)
def _make_flatten(b0: int, b1: int):
    wpb = _NW // b0
    bpb = b1 // wpb

    @functools.partial(
        pl.kernel,
        mesh=_MESH,
        out_type=jax.ShapeDtypeStruct((b0 * b1,), jnp.int32),
        scratch_types=[
            pltpu.VMEM((_BPW,), jnp.int32),
        ],
        compiler_params=pltpu.CompilerParams(use_tc_tiling_on_sc=True),
    )
    def flatten(x_hbm, xf_hbm, idx_v):
        wid = lax.axis_index("s") * _NC + lax.axis_index("c")
        b = wid // wpb
        off = (wid % wpb) * bpb
        for k in range(_BPW // _XT):
            pltpu.sync_copy(
                x_hbm.at[b, pl.ds(off + k * _XT, _XT)],
                idx_v.at[pl.ds(k * _XT, _XT)],
            )
        pltpu.sync_copy(idx_v, xf_hbm.at[pl.ds(b * b1 + off, _BPW)])

    return flatten


@functools.lru_cache(maxsize=None)
def _make_gather(b0: int, b1: int):
    wpb = _NW // b0
    bpb = b1 // wpb

    @functools.partial(
        pl.kernel,
        mesh=_MESH,
        out_type=jax.ShapeDtypeStruct((b0, b1, _D), jnp.float32),
        scratch_types=[
            pltpu.VMEM((_BPW,), jnp.int32),
            pltpu.VMEM((_BPW, _D), jnp.float32),
            pltpu.SemaphoreType.DMA,
        ],
        compiler_params=pltpu.CompilerParams(use_tc_tiling_on_sc=False),
    )
    def gather(xf_hbm, w_hbm, out_hbm, idx_v, rows_v, sem):
        wid = lax.axis_index("s") * _NC + lax.axis_index("c")
        b = wid // wpb
        off = (wid % wpb) * bpb
        pltpu.sync_copy(xf_hbm.at[pl.ds(b * b1 + off, _BPW)], idx_v)
        copies = [
            pltpu.async_copy(
                w_hbm.at[idx_v.at[pl.ds(c * _CHUNK, _CHUNK)]],
                rows_v.at[pl.ds(c * _CHUNK, _CHUNK)],
                sem,
            )
            for c in range(_NCHUNK)
        ]
        for h in copies:
            h.wait()
        pltpu.sync_copy(rows_v, out_hbm.at[b, pl.ds(off, _BPW)])

    return gather


def kernel(x, weight):
    b0, b1 = x.shape
    xf = _make_flatten(b0, b1)(x.astype(jnp.int32))
    return _make_gather(b0, b1)(xf, weight)


# speedup vs baseline: 1.0024x; 1.0005x over previous
import functools

import jax
import jax.numpy as jnp
from jax import lax
from jax.experimental import pallas as pl
from jax.experimental.pallas import tpu as pltpu
from jax.experimental.pallas import tpu_sc as plsc

_D = 64
_NC = 2
_NS = 16
_NW = _NC * _NS
_BPW = 1024
_CHUNK = 128
_NCHUNK = _BPW // _CHUNK

_MESH = plsc.VectorSubcoreMesh(core_axis_name="c", subcore_axis_name="s")


@functools.lru_cache(maxsize=None)
def _make_gather(b0: int, b1: int):
    wpb = _NW // b0
    bpb = b1 // wpb

    @functools.partial(
        pl.kernel,
        mesh=_MESH,
        out_type=jax.ShapeDtypeStruct((b0, b1, _D), jnp.float32),
        scratch_types=[
            pltpu.VMEM((_BPW,), jnp.float32),
            pltpu.VMEM((_BPW,), jnp.int32),
            pltpu.VMEM((_BPW, _D), jnp.float32),
            pltpu.SemaphoreType.DMA,
        ],
        compiler_params=pltpu.CompilerParams(
            use_tc_tiling_on_sc=False, needs_layout_passes=False
        ),
    )
    def gather(xf_hbm, w_hbm, out_hbm, idxf_v, idx_v, rows_v, sem):
        wid = lax.axis_index("s") * _NC + lax.axis_index("c")
        b = wid // wpb
        off = (wid % wpb) * bpb
        pltpu.sync_copy(xf_hbm.at[pl.ds(b * b1 + off, _BPW)], idxf_v)
        for i in range(_BPW // 16):
            idx_v[pl.ds(i * 16, 16)] = plsc.bitcast(
                idxf_v[pl.ds(i * 16, 16)], jnp.int32
            )
        copies = [
            pltpu.async_copy(
                w_hbm.at[idx_v.at[pl.ds(c * _CHUNK, _CHUNK)]],
                rows_v.at[pl.ds(c * _CHUNK, _CHUNK)],
                sem,
            )
            for c in range(_NCHUNK)
        ]
        for h in copies:
            h.wait()
        pltpu.sync_copy(rows_v, out_hbm.at[b, pl.ds(off, _BPW)])

    return gather


def kernel(x, weight):
    b0, b1 = x.shape
    xf = jax.lax.bitcast_convert_type(x.astype(jnp.int32), jnp.float32)
    return _make_gather(b0, b1)(xf.reshape(-1), weight)


# speedup vs baseline: 2.4191x; 2.4133x over previous
import functools

import jax
import jax.numpy as jnp
from jax import lax
from jax.experimental import pallas as pl
from jax.experimental.pallas import tpu as pltpu
from jax.experimental.pallas import tpu_sc as plsc

_D = 64
_NC = 2
_NS = 16
_NW = _NC * _NS
_BPW = 1024
_K = 64
_NCHUNK = _BPW // _K
_XT = 128


@functools.lru_cache(maxsize=None)
def _make_gather(b0: int, b1: int):
    wpb = _NW // b0
    bpb = b1 // wpb
    mesh = plsc.VectorSubcoreMesh(core_axis_name="c", subcore_axis_name="s")

    @functools.partial(
        pl.kernel,
        mesh=mesh,
        out_type=jax.ShapeDtypeStruct((b0, b1, _D), jnp.float32),
        scratch_types=[
            pltpu.VMEM((_BPW,), jnp.int32),
            pltpu.VMEM((2, _K, _D), jnp.float32),
            pltpu.SemaphoreType.DMA,
            pltpu.SemaphoreType.DMA,
            pltpu.SemaphoreType.DMA,
            pltpu.SemaphoreType.DMA,
        ],
        compiler_params=pltpu.CompilerParams(use_tc_tiling_on_sc=True),
    )
    def gather(x_hbm, w_hbm, out_hbm, idx_v, rows_v, g0, g1, o0, o1):
        wid = lax.axis_index("s") * _NC + lax.axis_index("c")
        b = wid // wpb
        off = (wid % wpb) * bpb
        for k in range(_BPW // _XT):
            pltpu.sync_copy(
                x_hbm.at[b, pl.ds(off + k * _XT, _XT)],
                idx_v.at[pl.ds(k * _XT, _XT)],
            )
        gsems = [g0, g1]
        osems = [o0, o1]

        def fire_gathers(c, buf):
            for j16 in range(_K // 16):
                v = idx_v[pl.ds(c * _K + j16 * 16, 16)]
                for l in range(16):
                    pltpu.async_copy(
                        w_hbm.at[v[l] >> 3, pl.ds(v[l] & 7, 1)],
                        rows_v.at[buf, pl.ds(j16 * 16 + l, 1)],
                        gsems[buf],
                    )

        def drain_gathers(buf):
            pltpu.make_async_copy(
                out_hbm.at[b, pl.ds(off, _K)], rows_v.at[buf], gsems[buf]
            ).wait()

        def fire_out(c, buf):
            pltpu.async_copy(
                rows_v.at[buf],
                out_hbm.at[b, pl.ds(off + c * _K, _K)],
                osems[buf],
            )

        def drain_out(buf):
            pltpu.make_async_copy(
                rows_v.at[buf],
                out_hbm.at[b, pl.ds(off, _K)],
                osems[buf],
            ).wait()

        def pair_body(g, carry):
            for buf in (0, 1):
                c = 2 * g + buf

                @pl.when(c >= 2)
                def _():
                    drain_out(buf)

                fire_gathers(c, buf)

                @pl.when(c >= 1)
                def _():
                    drain_gathers(1 - buf)
                    fire_out(c - 1, 1 - buf)

            return carry

        lax.fori_loop(0, _NCHUNK // 2, pair_body, 0)
        last = _NCHUNK - 1
        drain_gathers(last % 2)
        fire_out(last, last % 2)
        drain_out(1 - last % 2)
        drain_out(last % 2)

    return gather


def kernel(x, weight):
    b0, b1 = x.shape
    w3 = weight.reshape(weight.shape[0] // 8, 8, _D)
    return _make_gather(b0, b1)(x.astype(jnp.int32), w3)


# speedup vs baseline: 2.4421x; 1.0095x over previous
import functools

import jax
import jax.numpy as jnp
from jax import lax
from jax.experimental import pallas as pl
from jax.experimental.pallas import tpu as pltpu
from jax.experimental.pallas import tpu_sc as plsc

_D = 64
_NC = 2
_NS = 16
_NW = _NC * _NS
_BPW = 1024
_K = 128
_NCHUNK = _BPW // _K
_XT = 128


@functools.lru_cache(maxsize=None)
def _make_gather(b0: int, b1: int):
    wpb = _NW // b0
    bpb = b1 // wpb
    mesh = plsc.VectorSubcoreMesh(core_axis_name="c", subcore_axis_name="s")

    @functools.partial(
        pl.kernel,
        mesh=mesh,
        out_type=jax.ShapeDtypeStruct((b0, b1, _D), jnp.float32),
        scratch_types=[
            pltpu.VMEM((_BPW,), jnp.int32),
            pltpu.VMEM((2, _K, _D), jnp.float32),
            pltpu.SemaphoreType.DMA,
            pltpu.SemaphoreType.DMA,
            pltpu.SemaphoreType.DMA,
            pltpu.SemaphoreType.DMA,
        ],
        compiler_params=pltpu.CompilerParams(use_tc_tiling_on_sc=True),
    )
    def gather(x_hbm, w_hbm, out_hbm, idx_v, rows_v, g0, g1, o0, o1):
        wid = lax.axis_index("s") * _NC + lax.axis_index("c")
        b = wid // wpb
        off = (wid % wpb) * bpb
        for k in range(_BPW // _XT):
            pltpu.sync_copy(
                x_hbm.at[b, pl.ds(off + k * _XT, _XT)],
                idx_v.at[pl.ds(k * _XT, _XT)],
            )
        gsems = [g0, g1]
        osems = [o0, o1]

        def fire_gathers(c, buf):
            for j16 in range(_K // 16):
                v = idx_v[pl.ds(c * _K + j16 * 16, 16)]
                for l in range(16):
                    pltpu.async_copy(
                        w_hbm.at[v[l] >> 3, pl.ds(v[l] & 7, 1)],
                        rows_v.at[buf, pl.ds(j16 * 16 + l, 1)],
                        gsems[buf],
                    )

        def drain_gathers(buf):
            pltpu.make_async_copy(
                out_hbm.at[b, pl.ds(off, _K)], rows_v.at[buf], gsems[buf]
            ).wait()

        def fire_out(c, buf):
            pltpu.async_copy(
                rows_v.at[buf],
                out_hbm.at[b, pl.ds(off + c * _K, _K)],
                osems[buf],
            )

        def drain_out(buf):
            pltpu.make_async_copy(
                rows_v.at[buf],
                out_hbm.at[b, pl.ds(off, _K)],
                osems[buf],
            ).wait()

        def pair_body(g, carry):
            for buf in (0, 1):
                c = 2 * g + buf

                @pl.when(c >= 2)
                def _():
                    drain_out(buf)

                fire_gathers(c, buf)

                @pl.when(c >= 1)
                def _():
                    drain_gathers(1 - buf)
                    fire_out(c - 1, 1 - buf)

            return carry

        lax.fori_loop(0, _NCHUNK // 2, pair_body, 0)
        last = _NCHUNK - 1
        drain_gathers(last % 2)
        fire_out(last, last % 2)
        drain_out(1 - last % 2)
        drain_out(last % 2)

    return gather


def kernel(x, weight):
    b0, b1 = x.shape
    w3 = weight.reshape(weight.shape[0] // 8, 8, _D)
    return _make_gather(b0, b1)(x.astype(jnp.int32), w3)
